# baseline (speedup 1.0000x reference)
import jax
import jax.numpy as jnp
from jax.experimental import pallas as pl


def _log_softmax(x):
    m = jnp.max(x, axis=-1, keepdims=True)
    s = x - m
    return s - jnp.log(jnp.sum(jnp.exp(s), axis=-1, keepdims=True))


def _joint_kernel(tn_ref, pn_ref, out_ref):
    tn = _log_softmax(tn_ref[...])
    pn = _log_softmax(pn_ref[...])
    v = jax.lax.broadcasted_iota(jnp.int32, pn.shape, 1)
    pn = jnp.where(v == 0, 0.0, pn)
    out_ref[...] = pn[None, :, :] + tn[:, None, :]


def kernel(tn_out, pn_out):
    B, T, V = tn_out.shape
    _, U, _ = pn_out.shape
    Tb = 128
    grid = (B, T // Tb)
    return pl.pallas_call(
        _joint_kernel,
        grid=grid,
        in_specs=[
            pl.BlockSpec((None, Tb, V), lambda b, t: (b, t, 0)),
            pl.BlockSpec((None, U, V), lambda b, t: (b, 0, 0)),
        ],
        out_specs=pl.BlockSpec((None, Tb, U, V), lambda b, t: (b, t, 0, 0)),
        out_shape=jax.ShapeDtypeStruct((B, T, U, V), tn_out.dtype),
    )(tn_out, pn_out)
